# Initial kernel scaffold; baseline (speedup 1.0000x reference)
import jax
import jax.numpy as jnp
from jax.experimental import pallas as pl


def kernel(z, codebooks):
    raise NotImplementedError("write your pallas kernel here")


# speedup vs baseline: 1.4433x; 1.4433x over previous
import jax
import jax.numpy as jnp
from jax.experimental import pallas as pl
from jax.experimental.pallas import tpu as pltpu

B = 8
D = 128
T = 2048
NUM_Q = 4
K = 1024
COMMIT = 0.25
TBLK = 512
NT = T // TBLK
N = B * T
def _argmin0(scores):
    mi = jnp.min(scores, axis=0, keepdims=True)
    iota = jax.lax.broadcasted_iota(jnp.int32, scores.shape, 0)
    return jnp.min(jnp.where(scores == mi, iota, K), axis=0, keepdims=True)


def _vq_body(z_ref, cb_ref, cn_ref, zq_ref, stats_ref, counts_scr, sq_scr):
    b = pl.program_id(0)
    t = pl.program_id(1)

    @pl.when((b == 0) & (t == 0))
    def _init():
        counts_scr[...] = jnp.zeros_like(counts_scr)
        sq_scr[...] = jnp.zeros_like(sq_scr)

    x = z_ref[0]
    r = x
    acc = jnp.zeros_like(x)
    iota_full = jax.lax.broadcasted_iota(jnp.int32, (K, TBLK), 0)
    for q in range(NUM_Q):
        cb = cb_ref[q]
        cn = cn_ref[:, q:q + 1]
        t1 = jnp.sum(r * r, axis=0, keepdims=True)
        prod = jax.lax.dot_general(
            cb.astype(jnp.bfloat16), r.astype(jnp.bfloat16),
            (((1,), (0,)), ((), ())),
            preferred_element_type=jnp.float32)
        scores = (t1 - 2.0 * prod) + cn
        idx = _argmin0(scores)
        onehot = (iota_full == idx).astype(jnp.float32)
        quant = jax.lax.dot_general(
            cb, onehot, (((0,), (0,)), ((), ())),
            precision=jax.lax.Precision.HIGHEST,
            preferred_element_type=jnp.float32)
        u = quant - r
        qst = r + u
        r = r - qst
        acc = acc + qst
        counts_scr[:, q:q + 1] += jnp.sum(onehot, axis=1, keepdims=True)
        sq_scr[q:q + 1, :] += jnp.sum(u * u)
    zq_ref[0] = acc

    @pl.when((b == B - 1) & (t == NT - 1))
    def _finalize():
        counts = counts_scr[:, 0:NUM_Q]
        p = counts * (1.0 / N)
        ent = jnp.sum(p * jnp.log(p + 1e-10), axis=0, keepdims=True)
        perp = jnp.exp(-ent)
        stats_ref[0:1, 0:NUM_Q] = perp
        total = jnp.sum(sq_scr[0:NUM_Q, 0:1]) * ((1.0 + COMMIT) / (N * D))
        stats_ref[1:2, 0:1] = jnp.reshape(total, (1, 1))


def kernel(z, codebooks):
    cn = jnp.sum(codebooks * codebooks, axis=2)
    cn_pad = jnp.pad(jnp.transpose(cn), ((0, 0), (0, 128 - NUM_Q)))
    zq, stats = pl.pallas_call(
        _vq_body,
        grid=(B, NT),
        in_specs=[
            pl.BlockSpec((1, D, TBLK), lambda b, t: (b, 0, t)),
            pl.BlockSpec((NUM_Q, K, D), lambda b, t: (0, 0, 0)),
            pl.BlockSpec((K, 128), lambda b, t: (0, 0)),
        ],
        out_specs=[
            pl.BlockSpec((1, D, TBLK), lambda b, t: (b, 0, t)),
            pl.BlockSpec((8, 128), lambda b, t: (0, 0)),
        ],
        out_shape=[
            jax.ShapeDtypeStruct((B, D, T), jnp.float32),
            jax.ShapeDtypeStruct((8, 128), jnp.float32),
        ],
        scratch_shapes=[
            pltpu.VMEM((K, 128), jnp.float32),
            pltpu.VMEM((8, 128), jnp.float32),
        ],
    )(z, codebooks, cn_pad)
    return zq, stats[1, 0], stats[0, 0:NUM_Q]


# speedup vs baseline: 1.9691x; 1.3643x over previous
import jax
import jax.numpy as jnp
from jax.experimental import pallas as pl
from jax.experimental.pallas import tpu as pltpu

B = 8
D = 128
T = 2048
NUM_Q = 4
K = 1024
COMMIT = 0.25
TBLK = 512
NT = T // TBLK
N = B * T
def _argmin0(scores):
    mi = jnp.min(scores, axis=0, keepdims=True)
    iota = jax.lax.broadcasted_iota(jnp.int32, scores.shape, 0)
    return jnp.min(jnp.where(scores == mi, iota, K), axis=0, keepdims=True)


def _vq_body(z_ref, cb_ref, cn_ref, zq_ref, stats_ref, counts_scr, sq_scr):
    b = pl.program_id(0)
    t = pl.program_id(1)

    @pl.when((b == 0) & (t == 0))
    def _init():
        counts_scr[...] = jnp.zeros_like(counts_scr)
        sq_scr[...] = jnp.zeros_like(sq_scr)

    x = z_ref[0]
    r = x
    acc = jnp.zeros_like(x)
    iota_full = jax.lax.broadcasted_iota(jnp.int32, (K, TBLK), 0)
    for q in range(NUM_Q):
        cb = cb_ref[q]
        cn = cn_ref[:, q:q + 1]
        t1 = jnp.sum(r * r, axis=0, keepdims=True)
        prod = jax.lax.dot_general(
            cb.astype(jnp.bfloat16), r.astype(jnp.bfloat16),
            (((1,), (0,)), ((), ())),
            preferred_element_type=jnp.float32)
        scores = (t1 - 2.0 * prod) + cn
        idx = _argmin0(scores)
        onehot = (iota_full == idx).astype(jnp.bfloat16)
        cb_hi = cb.astype(jnp.bfloat16)
        rem = cb - cb_hi.astype(jnp.float32)
        cb_mid = rem.astype(jnp.bfloat16)
        cb_lo = (rem - cb_mid.astype(jnp.float32)).astype(jnp.bfloat16)
        dn = (((0,), (0,)), ((), ()))
        quant = (jax.lax.dot_general(cb_hi, onehot, dn,
                                     preferred_element_type=jnp.float32)
                 + jax.lax.dot_general(cb_mid, onehot, dn,
                                       preferred_element_type=jnp.float32)) \
            + jax.lax.dot_general(cb_lo, onehot, dn,
                                  preferred_element_type=jnp.float32)
        u = quant - r
        qst = r + u
        r = r - qst
        acc = acc + qst
        counts_scr[:, q:q + 1] += jnp.sum(onehot.astype(jnp.float32),
                                          axis=1, keepdims=True)
        sq_scr[q:q + 1, :] += jnp.sum(u * u)
    zq_ref[0] = acc

    @pl.when((b == B - 1) & (t == NT - 1))
    def _finalize():
        counts = counts_scr[:, 0:NUM_Q]
        p = counts * (1.0 / N)
        ent = jnp.sum(p * jnp.log(p + 1e-10), axis=0, keepdims=True)
        perp = jnp.exp(-ent)
        stats_ref[0:1, 0:NUM_Q] = perp
        total = jnp.sum(sq_scr[0:NUM_Q, 0:1]) * ((1.0 + COMMIT) / (N * D))
        stats_ref[1:2, 0:1] = jnp.reshape(total, (1, 1))


def kernel(z, codebooks):
    cn = jnp.sum(codebooks * codebooks, axis=2)
    cn_pad = jnp.pad(jnp.transpose(cn), ((0, 0), (0, 128 - NUM_Q)))
    zq, stats = pl.pallas_call(
        _vq_body,
        grid=(B, NT),
        in_specs=[
            pl.BlockSpec((1, D, TBLK), lambda b, t: (b, 0, t)),
            pl.BlockSpec((NUM_Q, K, D), lambda b, t: (0, 0, 0)),
            pl.BlockSpec((K, 128), lambda b, t: (0, 0)),
        ],
        out_specs=[
            pl.BlockSpec((1, D, TBLK), lambda b, t: (b, 0, t)),
            pl.BlockSpec((8, 128), lambda b, t: (0, 0)),
        ],
        out_shape=[
            jax.ShapeDtypeStruct((B, D, T), jnp.float32),
            jax.ShapeDtypeStruct((8, 128), jnp.float32),
        ],
        scratch_shapes=[
            pltpu.VMEM((K, 128), jnp.float32),
            pltpu.VMEM((8, 128), jnp.float32),
        ],
    )(z, codebooks, cn_pad)
    return zq, stats[1, 0], stats[0, 0:NUM_Q]


# speedup vs baseline: 2.3105x; 1.1734x over previous
import jax
import jax.numpy as jnp
from jax.experimental import pallas as pl
from jax.experimental.pallas import tpu as pltpu

B = 8
D = 128
T = 2048
NUM_Q = 4
K = 1024
COMMIT = 0.25
TBLK = 1024
NT = T // TBLK
N = B * T
def _argmin0(scores):
    mi = jnp.min(scores, axis=0, keepdims=True)
    iota = jax.lax.broadcasted_iota(jnp.int32, scores.shape, 0)
    return jnp.min(jnp.where(scores == mi, iota, K), axis=0, keepdims=True)


def _vq_body(z_ref, cb_ref, cn_ref, zq_ref, stats_ref, counts_scr, sq_scr):
    b = pl.program_id(0)
    t = pl.program_id(1)

    @pl.when((b == 0) & (t == 0))
    def _init():
        counts_scr[...] = jnp.zeros_like(counts_scr)
        sq_scr[...] = jnp.zeros_like(sq_scr)

    x = z_ref[0]
    r = x
    acc = jnp.zeros_like(x)
    iota_full = jax.lax.broadcasted_iota(jnp.int32, (K, TBLK), 0)
    for q in range(NUM_Q):
        cb = cb_ref[q]
        cn = cn_ref[:, q:q + 1]
        t1 = jnp.sum(r * r, axis=0, keepdims=True)
        prod = jax.lax.dot_general(
            cb.astype(jnp.bfloat16), r.astype(jnp.bfloat16),
            (((1,), (0,)), ((), ())),
            preferred_element_type=jnp.float32)
        scores = (t1 - 2.0 * prod) + cn
        idx = _argmin0(scores)
        onehot = (iota_full == idx).astype(jnp.bfloat16)
        cb_hi = cb.astype(jnp.bfloat16)
        rem = cb - cb_hi.astype(jnp.float32)
        cb_mid = rem.astype(jnp.bfloat16)
        cb_lo = (rem - cb_mid.astype(jnp.float32)).astype(jnp.bfloat16)
        dn = (((0,), (0,)), ((), ()))
        quant = (jax.lax.dot_general(cb_hi, onehot, dn,
                                     preferred_element_type=jnp.float32)
                 + jax.lax.dot_general(cb_mid, onehot, dn,
                                       preferred_element_type=jnp.float32)) \
            + jax.lax.dot_general(cb_lo, onehot, dn,
                                  preferred_element_type=jnp.float32)
        u = quant - r
        qst = r + u
        r = r - qst
        acc = acc + qst
        counts_scr[:, q:q + 1] += jnp.sum(onehot.astype(jnp.float32),
                                          axis=1, keepdims=True)
        sq_scr[q:q + 1, :] += jnp.sum(u * u)
    zq_ref[0] = acc

    @pl.when((b == B - 1) & (t == NT - 1))
    def _finalize():
        counts = counts_scr[:, 0:NUM_Q]
        p = counts * (1.0 / N)
        ent = jnp.sum(p * jnp.log(p + 1e-10), axis=0, keepdims=True)
        perp = jnp.exp(-ent)
        stats_ref[0:1, 0:NUM_Q] = perp
        total = jnp.sum(sq_scr[0:NUM_Q, 0:1]) * ((1.0 + COMMIT) / (N * D))
        stats_ref[1:2, 0:1] = jnp.reshape(total, (1, 1))


def kernel(z, codebooks):
    cn = jnp.sum(codebooks * codebooks, axis=2)
    cn_pad = jnp.pad(jnp.transpose(cn), ((0, 0), (0, 128 - NUM_Q)))
    zq, stats = pl.pallas_call(
        _vq_body,
        grid=(B, NT),
        in_specs=[
            pl.BlockSpec((1, D, TBLK), lambda b, t: (b, 0, t)),
            pl.BlockSpec((NUM_Q, K, D), lambda b, t: (0, 0, 0)),
            pl.BlockSpec((K, 128), lambda b, t: (0, 0)),
        ],
        out_specs=[
            pl.BlockSpec((1, D, TBLK), lambda b, t: (b, 0, t)),
            pl.BlockSpec((8, 128), lambda b, t: (0, 0)),
        ],
        out_shape=[
            jax.ShapeDtypeStruct((B, D, T), jnp.float32),
            jax.ShapeDtypeStruct((8, 128), jnp.float32),
        ],
        scratch_shapes=[
            pltpu.VMEM((K, 128), jnp.float32),
            pltpu.VMEM((8, 128), jnp.float32),
        ],
    )(z, codebooks, cn_pad)
    return zq, stats[1, 0], stats[0, 0:NUM_Q]


# speedup vs baseline: 2.3574x; 1.0203x over previous
import jax
import jax.numpy as jnp
from jax.experimental import pallas as pl
from jax.experimental.pallas import tpu as pltpu

B = 8
D = 128
T = 2048
NUM_Q = 4
K = 1024
COMMIT = 0.25
TBLK = 2048
NT = T // TBLK
N = B * T
def _argmin0(scores):
    mi = jnp.min(scores, axis=0, keepdims=True)
    iota = jax.lax.broadcasted_iota(jnp.int32, scores.shape, 0)
    return jnp.min(jnp.where(scores == mi, iota, K), axis=0, keepdims=True)


def _vq_body(z_ref, cb_ref, cn_ref, zq_ref, stats_ref, counts_scr, sq_scr):
    b = pl.program_id(0)
    t = pl.program_id(1)

    @pl.when((b == 0) & (t == 0))
    def _init():
        counts_scr[...] = jnp.zeros_like(counts_scr)
        sq_scr[...] = jnp.zeros_like(sq_scr)

    x = z_ref[0]
    r = x
    acc = jnp.zeros_like(x)
    iota_full = jax.lax.broadcasted_iota(jnp.int32, (K, TBLK), 0)
    for q in range(NUM_Q):
        cb = cb_ref[q]
        cn = cn_ref[:, q:q + 1]
        t1 = jnp.sum(r * r, axis=0, keepdims=True)
        prod = jax.lax.dot_general(
            cb.astype(jnp.bfloat16), r.astype(jnp.bfloat16),
            (((1,), (0,)), ((), ())),
            preferred_element_type=jnp.float32)
        scores = (t1 - 2.0 * prod) + cn
        idx = _argmin0(scores)
        onehot = (iota_full == idx).astype(jnp.bfloat16)
        cb_hi = cb.astype(jnp.bfloat16)
        rem = cb - cb_hi.astype(jnp.float32)
        cb_mid = rem.astype(jnp.bfloat16)
        cb_lo = (rem - cb_mid.astype(jnp.float32)).astype(jnp.bfloat16)
        dn = (((0,), (0,)), ((), ()))
        quant = (jax.lax.dot_general(cb_hi, onehot, dn,
                                     preferred_element_type=jnp.float32)
                 + jax.lax.dot_general(cb_mid, onehot, dn,
                                       preferred_element_type=jnp.float32)) \
            + jax.lax.dot_general(cb_lo, onehot, dn,
                                  preferred_element_type=jnp.float32)
        u = quant - r
        qst = r + u
        r = r - qst
        acc = acc + qst
        counts_scr[:, q:q + 1] += jnp.sum(onehot.astype(jnp.float32),
                                          axis=1, keepdims=True)
        sq_scr[q:q + 1, :] += jnp.sum(u * u)
    zq_ref[0] = acc

    @pl.when((b == B - 1) & (t == NT - 1))
    def _finalize():
        counts = counts_scr[:, 0:NUM_Q]
        p = counts * (1.0 / N)
        ent = jnp.sum(p * jnp.log(p + 1e-10), axis=0, keepdims=True)
        perp = jnp.exp(-ent)
        stats_ref[0:1, 0:NUM_Q] = perp
        total = jnp.sum(sq_scr[0:NUM_Q, 0:1]) * ((1.0 + COMMIT) / (N * D))
        stats_ref[1:2, 0:1] = jnp.reshape(total, (1, 1))


def kernel(z, codebooks):
    cn = jnp.sum(codebooks * codebooks, axis=2)
    cn_pad = jnp.pad(jnp.transpose(cn), ((0, 0), (0, 128 - NUM_Q)))
    zq, stats = pl.pallas_call(
        _vq_body,
        grid=(B, NT),
        in_specs=[
            pl.BlockSpec((1, D, TBLK), lambda b, t: (b, 0, t)),
            pl.BlockSpec((NUM_Q, K, D), lambda b, t: (0, 0, 0)),
            pl.BlockSpec((K, 128), lambda b, t: (0, 0)),
        ],
        out_specs=[
            pl.BlockSpec((1, D, TBLK), lambda b, t: (b, 0, t)),
            pl.BlockSpec((8, 128), lambda b, t: (0, 0)),
        ],
        out_shape=[
            jax.ShapeDtypeStruct((B, D, T), jnp.float32),
            jax.ShapeDtypeStruct((8, 128), jnp.float32),
        ],
        scratch_shapes=[
            pltpu.VMEM((K, 128), jnp.float32),
            pltpu.VMEM((8, 128), jnp.float32),
        ],
    )(z, codebooks, cn_pad)
    return zq, stats[1, 0], stats[0, 0:NUM_Q]


# speedup vs baseline: 2.3923x; 1.0148x over previous
import jax
import jax.numpy as jnp
from jax.experimental import pallas as pl
from jax.experimental.pallas import tpu as pltpu

B = 8
D = 128
T = 2048
NUM_Q = 4
K = 1024
COMMIT = 0.25
TBLK = 2048
NT = T // TBLK
N = B * T
def _argmin0(scores):
    mi = jnp.min(scores, axis=0, keepdims=True)
    iota = jax.lax.broadcasted_iota(jnp.int32, scores.shape, 0)
    return jnp.min(jnp.where(scores == mi, iota, K), axis=0, keepdims=True)


def _vq_body(z_ref, cb_ref, cn_ref, zq_ref, stats_ref, counts_scr, sq_scr,
             dec_scr):
    b = pl.program_id(0)
    t = pl.program_id(1)

    @pl.when((b == 0) & (t == 0))
    def _init():
        counts_scr[...] = jnp.zeros_like(counts_scr)
        sq_scr[...] = jnp.zeros_like(sq_scr)
        for q in range(NUM_Q):
            cb = cb_ref[q]
            hi = cb.astype(jnp.bfloat16)
            rem = cb - hi.astype(jnp.float32)
            mid = rem.astype(jnp.bfloat16)
            lo = (rem - mid.astype(jnp.float32)).astype(jnp.bfloat16)
            dec_scr[0, q] = hi
            dec_scr[1, q] = mid
            dec_scr[2, q] = lo

    x = z_ref[0]
    r = x
    acc = jnp.zeros_like(x)
    iota_full = jax.lax.broadcasted_iota(jnp.int32, (K, TBLK), 0)
    for q in range(NUM_Q):
        cn = cn_ref[:, q:q + 1]
        t1 = jnp.sum(r * r, axis=0, keepdims=True)
        cb_hi = dec_scr[0, q]
        prod = jax.lax.dot_general(
            cb_hi, r.astype(jnp.bfloat16),
            (((1,), (0,)), ((), ())),
            preferred_element_type=jnp.float32)
        scores = (t1 - 2.0 * prod) + cn
        idx = _argmin0(scores)
        onehot = (iota_full == idx).astype(jnp.bfloat16)
        cb_mid = dec_scr[1, q]
        cb_lo = dec_scr[2, q]
        dn = (((0,), (0,)), ((), ()))
        quant = (jax.lax.dot_general(cb_hi, onehot, dn,
                                     preferred_element_type=jnp.float32)
                 + jax.lax.dot_general(cb_mid, onehot, dn,
                                       preferred_element_type=jnp.float32)) \
            + jax.lax.dot_general(cb_lo, onehot, dn,
                                  preferred_element_type=jnp.float32)
        u = quant - r
        qst = r + u
        r = r - qst
        acc = acc + qst
        counts_scr[:, q:q + 1] += jnp.sum(onehot.astype(jnp.float32),
                                          axis=1, keepdims=True)
        sq_scr[q:q + 1, :] += jnp.sum(u * u)
    zq_ref[0] = acc

    @pl.when((b == B - 1) & (t == NT - 1))
    def _finalize():
        counts = counts_scr[:, 0:NUM_Q]
        p = counts * (1.0 / N)
        ent = jnp.sum(p * jnp.log(p + 1e-10), axis=0, keepdims=True)
        perp = jnp.exp(-ent)
        stats_ref[0:1, 0:NUM_Q] = perp
        total = jnp.sum(sq_scr[0:NUM_Q, 0:1]) * ((1.0 + COMMIT) / (N * D))
        stats_ref[1:2, 0:1] = jnp.reshape(total, (1, 1))


def kernel(z, codebooks):
    cn = jnp.sum(codebooks * codebooks, axis=2)
    cn_pad = jnp.pad(jnp.transpose(cn), ((0, 0), (0, 128 - NUM_Q)))
    zq, stats = pl.pallas_call(
        _vq_body,
        grid=(B, NT),
        in_specs=[
            pl.BlockSpec((1, D, TBLK), lambda b, t: (b, 0, t)),
            pl.BlockSpec((NUM_Q, K, D), lambda b, t: (0, 0, 0)),
            pl.BlockSpec((K, 128), lambda b, t: (0, 0)),
        ],
        out_specs=[
            pl.BlockSpec((1, D, TBLK), lambda b, t: (b, 0, t)),
            pl.BlockSpec((8, 128), lambda b, t: (0, 0)),
        ],
        out_shape=[
            jax.ShapeDtypeStruct((B, D, T), jnp.float32),
            jax.ShapeDtypeStruct((8, 128), jnp.float32),
        ],
        scratch_shapes=[
            pltpu.VMEM((K, 128), jnp.float32),
            pltpu.VMEM((8, 128), jnp.float32),
            pltpu.VMEM((3, NUM_Q, K, D), jnp.bfloat16),
        ],
    )(z, codebooks, cn_pad)
    return zq, stats[1, 0], stats[0, 0:NUM_Q]
